# Initial kernel scaffold; baseline (speedup 1.0000x reference)
import jax
import jax.numpy as jnp
from jax.experimental import pallas as pl


def kernel(x, edge_index, batch, W0, b0, W1, b1, W2, b2, g0, be0, g1, be1):
    raise NotImplementedError("write your pallas kernel here")


# speedup vs baseline: 8.3064x; 8.3064x over previous
import functools

import jax
import jax.numpy as jnp
from jax import lax
from jax.experimental import pallas as pl
from jax.experimental.pallas import tpu as pltpu
from jax.experimental.pallas import tpu_sc as plsc

_N, _E, _D = 10000, 320000, 128
_NP = 10240
_NC, _NS = 2, 16
_C = 128
_EP = 323584
_RPS = _NP // _NS
_BR = 512


def _sc_mesh():
    return plsc.VectorSubcoreMesh(
        core_axis_name="c", subcore_axis_name="s",
        num_cores=_NC, num_subcores=_NS)


def _deg_body(dst_hbm, ones_hbm, zeros_hbm, deg_hbm, acc, didx, ones_v):
    c = lax.axis_index("c")
    s = lax.axis_index("s")
    nchunks = _EP // _NS // _C

    @pl.when(c == 0)
    def _():
        pltpu.sync_copy(zeros_hbm.at[pl.ds(s * _RPS, _RPS)],
                        acc.at[pl.ds(s * _RPS, _RPS)])
        pltpu.sync_copy(ones_hbm, ones_v)
        plsc.subcore_barrier()

        def step(j, carry):
            base = s * (nchunks * _C) + j * _C
            pltpu.sync_copy(dst_hbm.at[pl.ds(base, _C)], didx)
            pltpu.sync_copy(ones_v, acc.at[didx], add=True)
            return carry

        lax.fori_loop(0, nchunks, step, 0)
        plsc.subcore_barrier()
        pltpu.sync_copy(acc.at[pl.ds(s * _RPS, _RPS)],
                        deg_hbm.at[pl.ds(s * _RPS, _RPS)])


def _deg_call(dstp, ones_hbm, zeros1):
    return pl.kernel(
        _deg_body,
        out_type=jax.ShapeDtypeStruct((_NP,), jnp.float32),
        mesh=_sc_mesh(),
        scratch_types=[
            pltpu.VMEM_SHARED((_NP,), jnp.float32),
            pltpu.VMEM((_C,), jnp.int32),
            pltpu.VMEM((_C,), jnp.float32),
        ],
    )(dstp, ones_hbm, zeros1)


def _prop_body(u_hbm, src_hbm, dst_hbm, out_hbm, acc, sidx, didx, rows, sem):
    c = lax.axis_index("c")
    s = lax.axis_index("s")
    pltpu.sync_copy(u_hbm.at[pl.ds(s * _RPS, _RPS)],
                    acc.at[pl.ds(s * _RPS, _RPS)])
    plsc.subcore_barrier()
    epw = _EP // (_NC * _NS)
    nchunks = epw // _C
    base_w = (c * _NS + s) * epw

    def step(j, carry):
        base = base_w + j * _C
        pltpu.sync_copy(src_hbm.at[pl.ds(base, _C)], sidx)
        pltpu.sync_copy(dst_hbm.at[pl.ds(base, _C)], didx)
        pltpu.async_copy(u_hbm.at[sidx], rows, sem).wait()
        pltpu.sync_copy(rows, acc.at[didx], add=True)
        return carry

    lax.fori_loop(0, nchunks, step, 0)
    plsc.subcore_barrier()
    pltpu.sync_copy(acc.at[pl.ds(s * _RPS, _RPS)],
                    out_hbm.at[pl.ds(c * _NP + s * _RPS, _RPS)])


def _prop_call(u, srcp, dstp):
    return pl.kernel(
        _prop_body,
        out_type=jax.ShapeDtypeStruct((_NC * _NP, _D), jnp.float32),
        mesh=_sc_mesh(),
        scratch_types=[
            pltpu.VMEM_SHARED((_NP, _D), jnp.float32),
            pltpu.VMEM((_C,), jnp.int32),
            pltpu.VMEM((_C,), jnp.int32),
            pltpu.VMEM((_C, _D), jnp.float32),
            pltpu.SemaphoreType.DMA,
        ],
    )(u, srcp, dstp)


def _tc0_body(deg_ref, x_ref, w_ref, u_ref):
    dis = lax.rsqrt(deg_ref[...] + 1.0)
    u_ref[...] = jnp.dot(x_ref[...], w_ref[...],
                         preferred_element_type=jnp.float32) * dis


def _tc0_call(deg2, xp, W0):
    grid = (_NP // _BR,)
    return pl.pallas_call(
        _tc0_body,
        grid=grid,
        in_specs=[
            pl.BlockSpec((_BR, 1), lambda i: (i, 0)),
            pl.BlockSpec((_BR, _D), lambda i: (i, 0)),
            pl.BlockSpec((_D, _D), lambda i: (0, 0)),
        ],
        out_specs=pl.BlockSpec((_BR, _D), lambda i: (i, 0)),
        out_shape=jax.ShapeDtypeStruct((_NP, _D), jnp.float32),
    )(deg2, xp, W0)


def _tcmid_body(deg_ref, s0_ref, s1_ref, up_ref, w_ref, b_ref, g_ref, be_ref,
                un_ref):
    dis = lax.rsqrt(deg_ref[...] + 1.0)
    z = (s0_ref[...] + s1_ref[...] - up_ref[...]) * dis + b_ref[...]
    h = jnp.maximum(z, 0.0)
    mu = jnp.mean(h, axis=-1, keepdims=True)
    d = h - mu
    var = jnp.mean(d * d, axis=-1, keepdims=True)
    hn = d * lax.rsqrt(var + 1e-5) * g_ref[...] + be_ref[...]
    un_ref[...] = jnp.dot(hn, w_ref[...],
                          preferred_element_type=jnp.float32) * dis


def _tcmid_call(deg2, sp, u_prev, Wn, bv, gv, bev):
    grid = (_NP // _BR,)
    nb = _NP // _BR
    return pl.pallas_call(
        _tcmid_body,
        grid=grid,
        in_specs=[
            pl.BlockSpec((_BR, 1), lambda i: (i, 0)),
            pl.BlockSpec((_BR, _D), lambda i: (i, 0)),
            pl.BlockSpec((_BR, _D), lambda i, nb=nb: (i + nb, 0)),
            pl.BlockSpec((_BR, _D), lambda i: (i, 0)),
            pl.BlockSpec((_D, _D), lambda i: (0, 0)),
            pl.BlockSpec((1, _D), lambda i: (0, 0)),
            pl.BlockSpec((1, _D), lambda i: (0, 0)),
            pl.BlockSpec((1, _D), lambda i: (0, 0)),
        ],
        out_specs=pl.BlockSpec((_BR, _D), lambda i: (i, 0)),
        out_shape=jax.ShapeDtypeStruct((_NP, _D), jnp.float32),
    )(deg2, sp, sp, u_prev, Wn, bv, gv, bev)


def _tcfin_body(deg_ref, s0_ref, s1_ref, up_ref, b_ref, emb_ref, h_ref):
    dis = lax.rsqrt(deg_ref[...] + 1.0)
    z = (s0_ref[...] + s1_ref[...] - up_ref[...]) * dis + b_ref[...]
    emb_ref[...] = z
    h_ref[...] = jnp.maximum(z, 0.0)


def _tcfin_call(deg2, sp, u_prev, bv):
    grid = (_NP // _BR,)
    nb = _NP // _BR
    return pl.pallas_call(
        _tcfin_body,
        grid=grid,
        in_specs=[
            pl.BlockSpec((_BR, 1), lambda i: (i, 0)),
            pl.BlockSpec((_BR, _D), lambda i: (i, 0)),
            pl.BlockSpec((_BR, _D), lambda i, nb=nb: (i + nb, 0)),
            pl.BlockSpec((_BR, _D), lambda i: (i, 0)),
            pl.BlockSpec((1, _D), lambda i: (0, 0)),
        ],
        out_specs=[
            pl.BlockSpec((_BR, _D), lambda i: (i, 0)),
            pl.BlockSpec((_BR, _D), lambda i: (i, 0)),
        ],
        out_shape=[
            jax.ShapeDtypeStruct((_NP, _D), jnp.float32),
            jax.ShapeDtypeStruct((_NP, _D), jnp.float32),
        ],
    )(deg2, sp, sp, u_prev, bv)


def kernel(x, edge_index, batch, W0, b0, W1, b1, W2, b2, g0, be0, g1, be1):
    f32 = jnp.float32
    xp = jnp.zeros((_NP, _D), f32).at[:_N].set(x)
    padidx = jnp.full((_EP - _E,), _NP - 1, jnp.int32)
    srcp = jnp.concatenate([edge_index[0].astype(jnp.int32), padidx])
    dstp = jnp.concatenate([edge_index[1].astype(jnp.int32), padidx])
    zeros1 = jnp.zeros((_NP,), f32)
    ones_c = jnp.ones((_C,), f32)

    deg = _deg_call(dstp, ones_c, zeros1)
    deg2 = deg.reshape(_NP, 1)

    b0v, b1v, b2v = (v.reshape(1, _D) for v in (b0, b1, b2))
    g0v, g1v = g0.reshape(1, _D), g1.reshape(1, _D)
    be0v, be1v = be0.reshape(1, _D), be1.reshape(1, _D)

    u0 = _tc0_call(deg2, xp, W0)
    sp0 = _prop_call(u0, srcp, dstp)
    u1 = _tcmid_call(deg2, sp0, u0, W1, b0v, g0v, be0v)
    sp1 = _prop_call(u1, srcp, dstp)
    u2 = _tcmid_call(deg2, sp1, u1, W2, b1v, g1v, be1v)
    sp2 = _prop_call(u2, srcp, dstp)
    emb, h = _tcfin_call(deg2, sp2, u2, b2v)
    return emb[:_N], h[:_N]
